# Initial kernel scaffold; baseline (speedup 1.0000x reference)
import jax
import jax.numpy as jnp
from jax.experimental import pallas as pl


def kernel(hidden_states, gate_weight, Wg, Wu, Wd, sWg, sWu, sWd):
    raise NotImplementedError("write your pallas kernel here")


# speedup vs baseline: 1.5064x; 1.5064x over previous
import functools

import jax
import jax.numpy as jnp
from jax import lax
from jax.experimental import pallas as pl

B, S, H = 2, 4096, 768
E, TOPK, FF = 16, 2, 384
SFF = 384 * 2
T = B * S
TM = 256
W_COLS = E * FF + SFF


def _moe_body(x_ref, gw_ref, w1_ref, w2_ref, w3_ref, p_ref, o_ref):
    x = x_ref[...]
    logits = lax.dot_general(x, gw_ref[...], (((1,), (1,)), ((), ())),
                             preferred_element_type=jnp.float32)
    m = jnp.max(logits, axis=-1, keepdims=True)
    p = jnp.exp(logits - m)
    s = p / jnp.sum(p, axis=-1, keepdims=True)
    iota = lax.broadcasted_iota(jnp.int32, (TM, E), 1)
    m1 = jnp.max(s, axis=-1, keepdims=True)
    idx1 = jnp.min(jnp.where(s == m1, iota, E), axis=-1, keepdims=True)
    oh1 = iota == idx1
    s2 = jnp.where(oh1, -1.0, s)
    m2 = jnp.max(s2, axis=-1, keepdims=True)
    idx2 = jnp.min(jnp.where(s2 == m2, iota, E), axis=-1, keepdims=True)
    oh2 = iota == idx2
    denom = m1 + m2 + 1e-20
    combine = jnp.where(oh1, m1 / denom, 0.0) + jnp.where(oh2, m2 / denom, 0.0)
    combine_ext = jnp.concatenate(
        [combine, jnp.ones((TM, 1), jnp.float32)], axis=1)
    scale = lax.dot_general(combine_ext, p_ref[...], (((1,), (0,)), ((), ())),
                            preferred_element_type=jnp.float32)
    xb = x.astype(jnp.bfloat16)
    g = lax.dot_general(xb, w1_ref[...], (((1,), (1,)), ((), ())),
                        preferred_element_type=jnp.float32)
    u = lax.dot_general(xb, w2_ref[...], (((1,), (1,)), ((), ())),
                        preferred_element_type=jnp.float32)
    a = ((g * jax.nn.sigmoid(g)) * u * scale).astype(jnp.bfloat16)
    o_ref[...] = lax.dot_general(a, w3_ref[...], (((1,), (0,)), ((), ())),
                                 preferred_element_type=jnp.float32)


@jax.jit
def _moe(x, gate_weight, w1, w2, w3, pexp):
    return pl.pallas_call(
        _moe_body,
        grid=(T // TM,),
        in_specs=[
            pl.BlockSpec((TM, H), lambda i: (i, 0)),
            pl.BlockSpec((E, H), lambda i: (0, 0)),
            pl.BlockSpec((W_COLS, H), lambda i: (0, 0)),
            pl.BlockSpec((W_COLS, H), lambda i: (0, 0)),
            pl.BlockSpec((W_COLS, H), lambda i: (0, 0)),
            pl.BlockSpec((E + 1, W_COLS), lambda i: (0, 0)),
        ],
        out_specs=pl.BlockSpec((TM, H), lambda i: (i, 0)),
        out_shape=jax.ShapeDtypeStruct((T, H), jnp.float32),
    )(x, gate_weight, w1, w2, w3, pexp)


def kernel(hidden_states, gate_weight, Wg, Wu, Wd, sWg, sWu, sWd):
    x = hidden_states.reshape(T, H)
    w1 = jnp.concatenate([Wg.reshape(E * FF, H), sWg], axis=0).astype(jnp.bfloat16)
    w2 = jnp.concatenate([Wu.reshape(E * FF, H), sWu], axis=0).astype(jnp.bfloat16)
    w3 = jnp.concatenate(
        [Wd.transpose(0, 2, 1).reshape(E * FF, H), sWd.T], axis=0).astype(jnp.bfloat16)
    col = jnp.arange(W_COLS) // FF
    pexp = (jnp.minimum(col, E)[None, :] == jnp.arange(E + 1)[:, None]
            ).astype(jnp.float32)
    y = _moe(x, gate_weight, w1, w2, w3, pexp)
    return y.reshape(B, S, H)
